# baseline (speedup 1.0000x reference)
import functools
import jax
import jax.numpy as jnp
from jax import lax
from jax.experimental import pallas as pl
from jax.experimental.pallas import tpu as pltpu

N = 10000
E = 160000
D = 256
H = 256
C = 64


def _k2_body(deg_ref, x_ref, rs_ref, xh_ref):
    deg = jnp.maximum(deg_ref[...], 1.0)
    rs = lax.rsqrt(deg)
    rs_ref[...] = rs
    xs = x_ref[...] * rs
    xh_ref[0] = xs[:, :128]
    xh_ref[1] = xs[:, 128:]


def _k2_prescale(deg, x):
    return pl.pallas_call(
        _k2_body,
        out_shape=(
            jax.ShapeDtypeStruct((N, 1), jnp.float32),
            jax.ShapeDtypeStruct((2, N, 128), jnp.float32),
        ),
    )(deg, x)


def _k4_body(rh_ref, rs_ref, w1_ref, b1_ref, w2_ref, qh_ref):
    rs = rs_ref[...]
    m = (jnp.dot(rh_ref[0], w1_ref[0, :128, :], preferred_element_type=jnp.float32)
         + jnp.dot(rh_ref[1], w1_ref[0, 128:, :], preferred_element_type=jnp.float32))
    pos_z = jax.nn.relu(rs * m + b1_ref[...])
    q = jnp.dot(pos_z, w2_ref[0], preferred_element_type=jnp.float32)
    qs = rs * q
    qh_ref[0] = qs[:, :32]
    qh_ref[1] = qs[:, 32:]


def _k4_dense(rh, rs, W1, b1, W2):
    bn = 2000
    grid = (N // bn,)
    return pl.pallas_call(
        _k4_body,
        grid=grid,
        in_specs=[
            pl.BlockSpec((2, bn, 128), lambda i: (0, i, 0)),
            pl.BlockSpec((bn, 1), lambda i: (i, 0)),
            pl.BlockSpec((1, H, H), lambda i: (0, 0, 0)),
            pl.BlockSpec((1, H), lambda i: (0, 0)),
            pl.BlockSpec((1, H, C), lambda i: (0, 0, 0)),
        ],
        out_specs=pl.BlockSpec((2, bn, 32), lambda i: (0, i, 0)),
        out_shape=jax.ShapeDtypeStruct((2, N, 32), jnp.float32),
    )(rh, rs, W1.reshape(1, H, H), b1.reshape(1, H), W2.reshape(1, H, C))


def _k6_body(r2_ref, rs_ref, b2_ref, o_ref):
    rs = rs_ref[...]
    o_ref[...] = rs * jnp.concatenate([r2_ref[0], r2_ref[1]], axis=1) + b2_ref[...]


def _k6_final(r2h, rs, b2):
    return pl.pallas_call(
        _k6_body,
        out_shape=jax.ShapeDtypeStruct((N, C), jnp.float32),
    )(r2h, rs, b2.reshape(1, C))


def _deg_hist(dst):
    return jnp.zeros((N,), jnp.float32).at[dst].add(1.0).reshape(N, 1)


def _spmm(halves, src, dst):
    f = halves.shape[-1]
    out0 = jnp.zeros((N, f), jnp.float32).at[dst].add(halves[0, src])
    out1 = jnp.zeros((N, f), jnp.float32).at[dst].add(halves[1, src])
    return jnp.stack([out0, out1])


def kernel(x, edge_index, perm, W1, b1, W2, b2):
    src, dst = edge_index[0], edge_index[1]
    deg = _deg_hist(dst)
    rs, xh = _k2_prescale(deg, x)
    r1 = _spmm(xh, src, dst)
    qh = _k4_dense(r1, rs, W1, b1, W2)
    r2 = _spmm(qh, src, dst)
    return _k6_final(r2, rs, b2)


# speedup vs baseline: 25.6954x; 25.6954x over previous
import functools
import jax
import jax.numpy as jnp
from jax import lax
from jax.experimental import pallas as pl
from jax.experimental.pallas import tpu as pltpu
from jax.experimental.pallas import tpu_sc as plsc

N = 10000
NP = 10240
E = 160000
D = 256
H = 256
C = 64

CH = 125
WIN = 8
ROWS = E // CH
NPT = NP // 16
RPT = ROWS // 16
RPTH = ROWS // 32

_MESH = plsc.VectorSubcoreMesh(core_axis_name="c", subcore_axis_name="s",
                               num_cores=2, num_subcores=16)


def _deg_body(dst_hbm, ones_hbm, zeros_hbm, out_hbm, dvm, ones_v, hist):
    cid = lax.axis_index("c")
    sid = lax.axis_index("s")
    pltpu.sync_copy(zeros_hbm.at[pl.ds(sid * NPT, NPT)],
                    hist.at[pl.ds(sid * NPT, NPT)])
    pltpu.sync_copy(ones_hbm.at[pl.ds(0, CH)], ones_v)
    base = cid * (16 * RPTH) + sid * RPTH
    pltpu.sync_copy(dst_hbm.at[pl.ds(base, RPTH)], dvm)
    plsc.subcore_barrier()

    def step(j, carry):
        pltpu.sync_copy(ones_v, hist.at[dvm.at[j]], add=True)
        return carry

    lax.fori_loop(0, RPTH, step, 0)
    plsc.subcore_barrier()
    pltpu.sync_copy(hist.at[pl.ds(sid * NPT, NPT)],
                    out_hbm.at[cid, pl.ds(sid * NPT, NPT)])


def _sc_deg(dst2d, ones, zeros_n):
    return pl.kernel(
        _deg_body,
        out_type=jax.ShapeDtypeStruct((2, NP), jnp.float32),
        mesh=_MESH,
        scratch_types=[
            pltpu.VMEM((RPTH, CH), jnp.int32),
            pltpu.VMEM((CH,), jnp.float32),
            pltpu.VMEM_SHARED((NP,), jnp.float32),
        ],
    )(dst2d, ones, zeros_n)


def _spmm_body(half_edges, xflat, srcs, dst_hbm, zeros_hbm, out_hbm,
               svm, dvm, rows0, rows1, acc, sem0, sem1):
    cid = lax.axis_index("c")
    sid = lax.axis_index("s")
    pltpu.sync_copy(zeros_hbm.at[pl.ds(sid * NPT, NPT)],
                    acc.at[pl.ds(sid * NPT, NPT)])
    if half_edges:
        rpt = RPTH
        rb = cid * (16 * RPTH) + sid * RPTH
    else:
        rpt = RPT
        rb = sid * RPT
    plsc.subcore_barrier()

    def window(w, carry):
        wb = rb + WIN * w
        if half_edges:
            pltpu.sync_copy(srcs.at[pl.ds(wb, WIN)], svm)
        else:
            pltpu.sync_copy(srcs.at[cid, pl.ds(wb, WIN)], svm)
        pltpu.sync_copy(dst_hbm.at[pl.ds(wb, WIN)], dvm)
        rows = (rows0, rows1)
        sems = (sem0, sem1)
        pltpu.async_copy(xflat.at[svm.at[0]], rows0, sem0)
        for j in range(WIN):
            if j + 1 < WIN:
                pltpu.async_copy(xflat.at[svm.at[j + 1]],
                                 rows[(j + 1) % 2], sems[(j + 1) % 2])
            pltpu.make_async_copy(xflat.at[svm.at[j]],
                                  rows[j % 2], sems[j % 2]).wait()
            pltpu.sync_copy(rows[j % 2], acc.at[dvm.at[j]], add=True)
        return carry

    lax.fori_loop(0, rpt // WIN, window, 0)
    plsc.subcore_barrier()
    pltpu.sync_copy(acc.at[pl.ds(sid * NPT, NPT)],
                    out_hbm.at[cid, pl.ds(sid * NPT, NPT)])


def _sc_spmm(xflat, srcs, dst2d, zeros, half_edges):
    return pl.kernel(
        functools.partial(_spmm_body, half_edges),
        out_type=jax.ShapeDtypeStruct((2, NP, 128), jnp.float32),
        mesh=_MESH,
        scratch_types=[
            pltpu.VMEM((WIN, CH), jnp.int32),
            pltpu.VMEM((WIN, CH), jnp.int32),
            pltpu.VMEM((CH, 128), jnp.float32),
            pltpu.VMEM((CH, 128), jnp.float32),
            pltpu.VMEM_SHARED((NP, 128), jnp.float32),
            pltpu.SemaphoreType.DMA,
            pltpu.SemaphoreType.DMA,
        ],
    )(xflat, srcs, dst2d, zeros)


def _k2_body(degp_ref, x_ref, rs_ref, xh_ref):
    deg = degp_ref[0] + degp_ref[1]
    rs = lax.rsqrt(jnp.maximum(deg, 1.0))
    rs_ref[...] = rs
    xs = x_ref[...] * rs[:N]
    xh_ref[0, :N] = xs[:, :128]
    xh_ref[1, :N] = xs[:, 128:]
    xh_ref[0, N:] = jnp.zeros((NP - N, 128), jnp.float32)
    xh_ref[1, N:] = jnp.zeros((NP - N, 128), jnp.float32)


def _k2_prescale(deg_parts, x):
    return pl.pallas_call(
        _k2_body,
        out_shape=(
            jax.ShapeDtypeStruct((NP, 1), jnp.float32),
            jax.ShapeDtypeStruct((2, NP, 128), jnp.float32),
        ),
    )(deg_parts, x)


def _k4_body(rh_ref, rs_ref, w1_ref, b1_ref, w2_ref, qh_ref):
    rs = rs_ref[...]
    m = (jnp.dot(rh_ref[0], w1_ref[0, :128, :], preferred_element_type=jnp.float32)
         + jnp.dot(rh_ref[1], w1_ref[0, 128:, :], preferred_element_type=jnp.float32))
    pos_z = jax.nn.relu(rs * m + b1_ref[...])
    q = jnp.dot(pos_z, w2_ref[0], preferred_element_type=jnp.float32)
    bn = q.shape[0]
    qh_ref[...] = jnp.concatenate(
        [rs * q, jnp.zeros((bn, 128 - C), jnp.float32)], axis=1)


def _k4_dense(rh, rs, W1, b1, W2):
    bn = 1280
    grid = (NP // bn,)
    return pl.pallas_call(
        _k4_body,
        grid=grid,
        in_specs=[
            pl.BlockSpec((2, bn, 128), lambda i: (0, i, 0)),
            pl.BlockSpec((bn, 1), lambda i: (i, 0)),
            pl.BlockSpec((1, H, H), lambda i: (0, 0, 0)),
            pl.BlockSpec((1, H), lambda i: (0, 0)),
            pl.BlockSpec((1, H, C), lambda i: (0, 0, 0)),
        ],
        out_specs=pl.BlockSpec((bn, 128), lambda i: (i, 0)),
        out_shape=jax.ShapeDtypeStruct((NP, 128), jnp.float32),
    )(rh, rs, W1.reshape(1, H, H), b1.reshape(1, H), W2.reshape(1, H, C))


def _k6_body(r2_ref, rs_ref, b2_ref, o_ref):
    rs = rs_ref[:N]
    o_ref[...] = (rs * (r2_ref[0, :N, :C] + r2_ref[1, :N, :C])
                  + b2_ref[...])


def _k6_final(r2h, rs, b2):
    return pl.pallas_call(
        _k6_body,
        out_shape=jax.ShapeDtypeStruct((N, C), jnp.float32),
    )(r2h, rs, b2.reshape(1, C))


def kernel(x, edge_index, perm, W1, b1, W2, b2):
    src, dst = edge_index[0], edge_index[1]
    dst2d = dst.reshape(ROWS, CH)
    src2d = src.reshape(ROWS, CH)
    srcs = jnp.stack([src, src + NP]).reshape(2, ROWS, CH)
    ones = jnp.ones((128,), jnp.float32)
    zeros_n = jnp.zeros((NP,), jnp.float32)
    zeros1 = jnp.zeros((NP, 128), jnp.float32)

    deg_parts = _sc_deg(dst2d, ones, zeros_n)
    rs, xh = _k2_prescale(deg_parts.reshape(2, NP, 1), x)
    r1 = _sc_spmm(xh.reshape(2 * NP, 128), srcs, dst2d, zeros1, False)
    qh = _k4_dense(r1, rs, W1, b1, W2)
    r2 = _sc_spmm(qh, src2d, dst2d, zeros1, True)
    return _k6_final(r2, rs, b2)


# speedup vs baseline: 29.0651x; 1.1311x over previous
import functools
import jax
import jax.numpy as jnp
from jax import lax
from jax.experimental import pallas as pl
from jax.experimental.pallas import tpu as pltpu
from jax.experimental.pallas import tpu_sc as plsc

N = 10000
NP = 10240
E = 160000
D = 256
H = 256
C = 64

CH = 125
WIN = 8
ROWS = E // CH
NPT = NP // 16
RPT = ROWS // 16
RPTH = ROWS // 32

_MESH = plsc.VectorSubcoreMesh(core_axis_name="c", subcore_axis_name="s",
                               num_cores=2, num_subcores=16)


def _deg_body(dst_hbm, ones_hbm, zeros_hbm, out_hbm, dvm, ones_v, hist):
    cid = lax.axis_index("c")
    sid = lax.axis_index("s")
    pltpu.sync_copy(zeros_hbm.at[pl.ds(sid * NPT, NPT)],
                    hist.at[pl.ds(sid * NPT, NPT)])
    pltpu.sync_copy(ones_hbm.at[pl.ds(0, CH)], ones_v)
    base = cid * (16 * RPTH) + sid * RPTH
    pltpu.sync_copy(dst_hbm.at[pl.ds(base, RPTH)], dvm)
    plsc.subcore_barrier()

    def step(j, carry):
        pltpu.sync_copy(ones_v, hist.at[dvm.at[j]], add=True)
        return carry

    lax.fori_loop(0, RPTH, step, 0)
    plsc.subcore_barrier()
    pltpu.sync_copy(hist.at[pl.ds(sid * NPT, NPT)],
                    out_hbm.at[cid, pl.ds(sid * NPT, NPT)])


def _sc_deg(dst2d, ones, zeros_n):
    return pl.kernel(
        _deg_body,
        out_type=jax.ShapeDtypeStruct((2, NP), jnp.float32),
        mesh=_MESH,
        scratch_types=[
            pltpu.VMEM((RPTH, CH), jnp.int32),
            pltpu.VMEM((CH,), jnp.float32),
            pltpu.VMEM_SHARED((NP,), jnp.float32),
        ],
    )(dst2d, ones, zeros_n)


def _spmm_body(half_edges, xflat, srcs, dst_hbm, zeros_hbm, out_hbm,
               svm0, dvm0, svm1, dvm1, rows0, rows1, acc,
               sem0, sem1, isem0, isem1):
    cid = lax.axis_index("c")
    sid = lax.axis_index("s")
    pltpu.sync_copy(zeros_hbm.at[pl.ds(sid * NPT, NPT)],
                    acc.at[pl.ds(sid * NPT, NPT)])
    if half_edges:
        rpt = RPTH
        rb = cid * (16 * RPTH) + sid * RPTH
    else:
        rpt = RPT
        rb = sid * RPT
    plsc.subcore_barrier()

    nw = rpt // WIN
    rows = (rows0, rows1)
    sems = (sem0, sem1)
    svm = (svm0, svm1)
    dvm = (dvm0, dvm1)
    isem = (isem0, isem1)

    def stage(w, b):
        wb = rb + WIN * w
        if half_edges:
            pltpu.async_copy(srcs.at[pl.ds(wb, WIN)], svm[b], isem[b])
        else:
            pltpu.async_copy(srcs.at[cid, pl.ds(wb, WIN)], svm[b], isem[b])
        pltpu.async_copy(dst_hbm.at[pl.ds(wb, WIN)], dvm[b], isem[b])

    def wait_stage(b):
        pltpu.make_async_copy(dst_hbm.at[pl.ds(rb, WIN)], svm[b], isem[b]).wait()
        pltpu.make_async_copy(dst_hbm.at[pl.ds(rb, WIN)], dvm[b], isem[b]).wait()

    def do_window(b, next_pred):
        for j in range(WIN):
            p = j % 2
            if j + 1 < WIN:
                pltpu.async_copy(xflat.at[svm[b].at[j + 1]],
                                 rows[1 - p], sems[1 - p])
            else:
                @pl.when(next_pred)
                def _():
                    wait_stage(1 - b)
                    pltpu.async_copy(xflat.at[svm[1 - b].at[0]],
                                     rows[1 - p], sems[1 - p])
            pltpu.make_async_copy(xflat.at[svm[b].at[j]],
                                  rows[p], sems[p]).wait()
            pltpu.sync_copy(rows[p], acc.at[dvm[b].at[j]], add=True)

    stage(0, 0)
    wait_stage(0)
    pltpu.async_copy(xflat.at[svm[0].at[0]], rows0, sem0)

    def pair(k, carry):
        w0 = 2 * k
        stage(w0 + 1, 1)
        do_window(0, w0 + 1 < nw)

        @pl.when(w0 + 2 < nw)
        def _():
            stage(w0 + 2, 0)

        do_window(1, w0 + 2 < nw)
        return carry

    lax.fori_loop(0, nw // 2, pair, 0)
    if nw % 2:
        do_window(0, jnp.bool_(False))
    plsc.subcore_barrier()
    pltpu.sync_copy(acc.at[pl.ds(sid * NPT, NPT)],
                    out_hbm.at[cid, pl.ds(sid * NPT, NPT)])


def _sc_spmm(xflat, srcs, dst2d, zeros, half_edges):
    return pl.kernel(
        functools.partial(_spmm_body, half_edges),
        out_type=jax.ShapeDtypeStruct((2, NP, 128), jnp.float32),
        mesh=_MESH,
        scratch_types=[
            pltpu.VMEM((WIN, CH), jnp.int32),
            pltpu.VMEM((WIN, CH), jnp.int32),
            pltpu.VMEM((WIN, CH), jnp.int32),
            pltpu.VMEM((WIN, CH), jnp.int32),
            pltpu.VMEM((CH, 128), jnp.float32),
            pltpu.VMEM((CH, 128), jnp.float32),
            pltpu.VMEM_SHARED((NP, 128), jnp.float32),
            pltpu.SemaphoreType.DMA,
            pltpu.SemaphoreType.DMA,
            pltpu.SemaphoreType.DMA,
            pltpu.SemaphoreType.DMA,
        ],
    )(xflat, srcs, dst2d, zeros)


def _k2_body(degp_ref, x_ref, rs_ref, xh_ref):
    deg = degp_ref[0] + degp_ref[1]
    rs = lax.rsqrt(jnp.maximum(deg, 1.0))
    rs_ref[...] = rs
    xs = x_ref[...] * rs[:N]
    xh_ref[0, :N] = xs[:, :128]
    xh_ref[1, :N] = xs[:, 128:]
    xh_ref[0, N:] = jnp.zeros((NP - N, 128), jnp.float32)
    xh_ref[1, N:] = jnp.zeros((NP - N, 128), jnp.float32)


def _k2_prescale(deg_parts, x):
    return pl.pallas_call(
        _k2_body,
        out_shape=(
            jax.ShapeDtypeStruct((NP, 1), jnp.float32),
            jax.ShapeDtypeStruct((2, NP, 128), jnp.float32),
        ),
    )(deg_parts, x)


def _k4_body(rh_ref, rs_ref, w1_ref, b1_ref, w2_ref, qh_ref):
    rs = rs_ref[...]
    m = (jnp.dot(rh_ref[0], w1_ref[0, :128, :], preferred_element_type=jnp.float32)
         + jnp.dot(rh_ref[1], w1_ref[0, 128:, :], preferred_element_type=jnp.float32))
    pos_z = jax.nn.relu(rs * m + b1_ref[...])
    q = jnp.dot(pos_z, w2_ref[0], preferred_element_type=jnp.float32)
    bn = q.shape[0]
    qh_ref[...] = jnp.concatenate(
        [rs * q, jnp.zeros((bn, 128 - C), jnp.float32)], axis=1)


def _k4_dense(rh, rs, W1, b1, W2):
    bn = 1280
    grid = (NP // bn,)
    return pl.pallas_call(
        _k4_body,
        grid=grid,
        in_specs=[
            pl.BlockSpec((2, bn, 128), lambda i: (0, i, 0)),
            pl.BlockSpec((bn, 1), lambda i: (i, 0)),
            pl.BlockSpec((1, H, H), lambda i: (0, 0, 0)),
            pl.BlockSpec((1, H), lambda i: (0, 0)),
            pl.BlockSpec((1, H, C), lambda i: (0, 0, 0)),
        ],
        out_specs=pl.BlockSpec((bn, 128), lambda i: (i, 0)),
        out_shape=jax.ShapeDtypeStruct((NP, 128), jnp.float32),
    )(rh, rs, W1.reshape(1, H, H), b1.reshape(1, H), W2.reshape(1, H, C))


def _k6_body(r2_ref, rs_ref, b2_ref, o_ref):
    rs = rs_ref[:N]
    o_ref[...] = (rs * (r2_ref[0, :N, :C] + r2_ref[1, :N, :C])
                  + b2_ref[...])


def _k6_final(r2h, rs, b2):
    return pl.pallas_call(
        _k6_body,
        out_shape=jax.ShapeDtypeStruct((N, C), jnp.float32),
    )(r2h, rs, b2.reshape(1, C))


def kernel(x, edge_index, perm, W1, b1, W2, b2):
    src, dst = edge_index[0], edge_index[1]
    dst2d = dst.reshape(ROWS, CH)
    src2d = src.reshape(ROWS, CH)
    srcs = jnp.stack([src, src + NP]).reshape(2, ROWS, CH)
    ones = jnp.ones((128,), jnp.float32)
    zeros_n = jnp.zeros((NP,), jnp.float32)
    zeros1 = jnp.zeros((NP, 128), jnp.float32)

    deg_parts = _sc_deg(dst2d, ones, zeros_n)
    rs, xh = _k2_prescale(deg_parts.reshape(2, NP, 1), x)
    r1 = _sc_spmm(xh.reshape(2 * NP, 128), srcs, dst2d, zeros1, False)
    qh = _k4_dense(r1, rs, W1, b1, W2)
    r2 = _sc_spmm(qh, src2d, dst2d, zeros1, True)
    return _k6_final(r2, rs, b2)
